# baseline (speedup 1.0000x reference)
import functools

import jax
import jax.numpy as jnp
from jax import lax
from jax.experimental import pallas as pl
from jax.experimental.pallas import tpu as pltpu
from jax.experimental.pallas import tpu_sc as plsc

NH = 8
DH = 16
L = 16
NC = 2
NS = 16
NW = NC * NS
EB = 128
EBA = 64

_GDN = lax.GatherDimensionNumbers(
    offset_dims=(), collapsed_slice_dims=(0,), start_index_map=(0,))


def _vperm(v, idx):
    return lax.gather(v, idx[:, None], _GDN, (1,),
                      mode=lax.GatherScatterMode.PROMISE_IN_BOUNDS)


def _sc_mesh():
    return plsc.VectorSubcoreMesh(core_axis_name="c", subcore_axis_name="s")


def _wid():
    return lax.axis_index("s") * NC + lax.axis_index("c")


def _qkv_body(x_ref, w_ref, b_ref, q_ref, k_ref, v_ref):
    acc = jnp.dot(x_ref[...], w_ref[...], preferred_element_type=jnp.float32)
    acc = acc + b_ref[...]
    q_ref[...] = acc[:, 0:128]
    k_ref[...] = acc[:, 128:256]
    v_ref[...] = acc[:, 256:384]


def _gate_body(x_ref, w_ref, b_ref, g_ref):
    g_ref[...] = jnp.dot(x_ref[...], w_ref[...],
                         preferred_element_type=jnp.float32) + b_ref[...]


def _oe_body(x_ref, s_ref, we_ref, be_ref, w_ref, b_ref, o_ref):
    ep = jnp.dot(x_ref[...], we_ref[...],
                 preferred_element_type=jnp.float32) + be_ref[...]
    rows = lax.broadcasted_iota(jnp.int32, (NH, 128), 0)
    cols = lax.broadcasted_iota(jnp.int32, (NH, 128), 1)
    expand = (cols // DH == rows).astype(jnp.float32)
    sx = jnp.dot(s_ref[...], expand, preferred_element_type=jnp.float32)
    o_ref[...] = jnp.dot(ep * sx, w_ref[...],
                         preferred_element_type=jnp.float32) + b_ref[...]


def _oh_body(a_ref, b2_ref, d0_ref, d1_ref, w_ref, b_ref, o_ref):
    h = a_ref[...] + b2_ref[...]
    den = d0_ref[...] + d1_ref[...]
    rows = lax.broadcasted_iota(jnp.int32, (NH, 128), 0)
    cols = lax.broadcasted_iota(jnp.int32, (NH, 128), 1)
    expand = (cols // DH == rows).astype(jnp.float32)
    dex = jnp.dot(den, expand, preferred_element_type=jnp.float32)
    dex = jnp.where(dex == 0.0, 1.0, dex)
    o_ref[...] = jnp.dot(h / dex, w_ref[...],
                         preferred_element_type=jnp.float32) + b_ref[...]


def _sddmm_body(nblk_total, qp, kp, row_h, col_h, eg_h, sc_h, wmax_h,
                row_v0, col_v0, q_v0, k_v0, eg_v0, sc_v0,
                row_v1, col_v1, q_v1, k_v1, eg_v1, sc_v1,
                mx_v, semg0, semg1, semw0, semw1):
    wid = _wid()
    lane = lax.iota(jnp.int32, L)
    p8 = lane ^ 8
    p4 = lane ^ 4
    p2 = lane ^ 2
    p1 = lane ^ 1
    lo8 = lane < 8
    lane8 = lane & 7

    nblk = nblk_total // NW + jnp.where(wid < nblk_total % NW, 1, 0)
    bufs = ((row_v0, col_v0, q_v0, k_v0, eg_v0, sc_v0, semg0, semw0),
            (row_v1, col_v1, q_v1, k_v1, eg_v1, sc_v1, semg1, semw1))

    mx_v[...] = jnp.full((L,), -3.4e38, jnp.float32)

    def fetch(b, blk):
        row_v, col_v, q_v, k_v, eg_v, sc_v, semg, semw = bufs[b]
        base = (wid + NW * blk) * EB
        pltpu.sync_copy(row_h.at[pl.ds(base, EB)], row_v)
        pltpu.sync_copy(col_h.at[pl.ds(base, EB)], col_v)
        pltpu.async_copy(qp.at[row_v], q_v, semg)
        pltpu.async_copy(kp.at[col_v], k_v, semg)
        pltpu.sync_copy(eg_h.at[pl.ds(base * NH, EB * NH)], eg_v)

    def compute(b, blk):
        row_v, col_v, q_v, k_v, eg_v, sc_v, semg, semw = bufs[b]
        base = (wid + NW * blk) * EB
        pltpu.make_async_copy(qp.at[row_v], q_v, semg).wait()
        pltpu.make_async_copy(kp.at[col_v], k_v, semg).wait()

        def pair(i, mx):
            mf = []
            for h in range(NH):
                p0 = q_v[2 * i, pl.ds(DH * h, L)] * k_v[2 * i, pl.ds(DH * h, L)]
                p1h = (q_v[2 * i + 1, pl.ds(DH * h, L)]
                       * k_v[2 * i + 1, pl.ds(DH * h, L)])
                a0 = p0 + _vperm(p0, p8)
                a1 = p1h + _vperm(p1h, p8)
                m = jnp.where(lo8, a0, a1)
                m = m + _vperm(m, p4)
                m = m + _vperm(m, p2)
                m = m + _vperm(m, p1)
                mf.append(m)
            s = mf[0]
            for h in range(1, NH):
                s = jnp.where(lane8 == h, mf[h], s)
            s = s * eg_v[pl.ds(L * i, L)]
            sc_v[pl.ds(L * i, L)] = s
            return jnp.maximum(mx, s)

        mx = lax.fori_loop(0, EB // 2, pair, mx_v[...])
        mx_v[...] = mx
        pltpu.async_copy(sc_v, sc_h.at[pl.ds(base * NH, EB * NH)], semw)

    def waitw(b):
        row_v, col_v, q_v, k_v, eg_v, sc_v, semg, semw = bufs[b]
        pltpu.make_async_copy(sc_v, sc_h.at[pl.ds(0, EB * NH)], semw).wait()

    fetch(0, 0)

    def pairstep(j2, _):
        for b in range(2):
            blk = 2 * j2 + b

            @pl.when(blk + 1 < nblk)
            def _(b=b, blk=blk):
                fetch(1 - b, blk + 1)

            @pl.when((blk >= 2) & (blk < nblk))
            def _(b=b):
                waitw(b)

            @pl.when(blk < nblk)
            def _(b=b, blk=blk):
                compute(b, blk)
        return 0

    lax.fori_loop(0, (nblk + 1) // 2, pairstep, 0)
    waitw(0)
    waitw(1)
    pltpu.sync_copy(mx_v, wmax_h.at[pl.ds(wid * L, L)])


def _accum_body(nblk_total, sc_h, wmax_h, row_h, col_h, vp, denp_h, hpart_h,
                sc_va, row_va, col_va, v_va, st_va, exs_va, eidx_va,
                sc_vb, row_vb, col_vb, v_vb, st_vb, exs_vb, eidx_vb,
                id_v, dn_v, wm_v, den_sh, h_sh,
                semg0, semg1, semh0, semh1, semd0, semd1):
    cid = lax.axis_index("c")
    sid = lax.axis_index("s")
    wid = sid * NC + cid
    npad = h_sh.shape[0]
    nper = npad // NS
    eper = nper * NH
    lane = lax.iota(jnp.int32, L)
    p8 = lane ^ 8
    lo8 = lane < 8
    lane8 = lane & 7

    nblk = nblk_total // NW + jnp.where(wid < nblk_total % NW, 1, 0)
    bufs = ((sc_va, row_va, col_va, v_va, st_va, exs_va, eidx_va,
             semg0, semh0, semd0),
            (sc_vb, row_vb, col_vb, v_vb, st_vb, exs_vb, eidx_vb,
             semg1, semh1, semd1))

    def fill_ids(base):
        for j in range(EBA // L):
            id_v[pl.ds(16 * j, L)] = base + 16 * j + lane

    def zbody(i, _):
        for h in range(NH):
            st_va[i, pl.ds(DH * h, L)] = jnp.zeros((L,), jnp.float32)
        return 0
    lax.fori_loop(0, EBA, zbody, 0)

    def zd(i, _):
        dn_v[pl.ds(L * i, L)] = jnp.zeros((L,), jnp.float32)
        return 0
    lax.fori_loop(0, EBA // L, zd, 0)

    for t in range(nper // EBA):
        fill_ids(sid * nper + t * EBA)
        pltpu.sync_copy(st_va, h_sh.at[id_v])

    def zden(k, _):
        fill_ids(sid * eper + k * EBA)
        pltpu.sync_copy(dn_v, den_sh.at[id_v])
        return 0
    lax.fori_loop(0, eper // EBA, zden, 0)
    plsc.subcore_barrier()

    pltpu.sync_copy(wmax_h, wm_v)
    m = wm_v[pl.ds(0, L)]
    for j in range(1, NW):
        m = jnp.maximum(m, wm_v[pl.ds(j * L, L)])
    m = jnp.maximum(m, _vperm(m, p8))

    def wait_scatters(b):
        sc_v, row_v, col_v, v_v, st_v, exs_v, eidx_v, semg, semh, semd = bufs[b]
        pltpu.make_async_copy(st_v, h_sh.at[row_v], semh).wait()
        for t in range(NH):
            pltpu.make_async_copy(exs_v.at[t],
                                  den_sh.at[eidx_v.at[t]], semd).wait()

    def fetch(b, blk):
        sc_v, row_v, col_v, v_v, st_v, exs_v, eidx_v, semg, semh, semd = bufs[b]
        base = (wid + NW * blk) * EBA

        @pl.when(blk >= 2)
        def _():
            wait_scatters(b)

        pltpu.sync_copy(col_h.at[pl.ds(base, EBA)], col_v)
        pltpu.sync_copy(row_h.at[pl.ds(base, EBA)], row_v)
        pltpu.async_copy(vp.at[col_v], v_v, semg)
        pltpu.sync_copy(sc_h.at[pl.ds(base * NH, EBA * NH)], sc_v)

    def compute(b, blk):
        sc_v, row_v, col_v, v_v, st_v, exs_v, eidx_v, semg, semh, semd = bufs[b]
        pltpu.make_async_copy(vp.at[col_v], v_v, semg).wait()

        def grp(g, _):
            rv16 = row_v[pl.ds(L * g, L)]
            for j2 in range(8):
                i = 8 * g + j2
                sv = sc_v[pl.ds(L * i, L)]
                attn = jnp.exp(sv - m)
                exs_v[i // 4, pl.ds(L * (i % 4), L)] = attn
                r0s = _vperm(rv16, jnp.full((L,), 2 * j2, jnp.int32))
                r1s = _vperm(rv16, jnp.full((L,), 2 * j2 + 1, jnp.int32))
                eidx_v[i // 4, pl.ds(L * (i % 4), L)] = (
                    jnp.where(lo8, r0s, r1s) * NH + lane8)
                for e01 in range(2):
                    for h in range(NH):
                        splat = _vperm(attn,
                                       jnp.full((L,), 8 * e01 + h, jnp.int32))
                        st_v[2 * i + e01, pl.ds(DH * h, L)] = (
                            v_v[2 * i + e01, pl.ds(DH * h, L)] * splat)
            return 0

        lax.fori_loop(0, EBA // L, grp, 0)
        pltpu.async_copy(st_v, h_sh.at[row_v], semh, add=True)
        for t in range(NH):
            pltpu.async_copy(exs_v.at[t], den_sh.at[eidx_v.at[t]],
                             semd, add=True)

    fetch(0, 0)

    def pairstep(j2, _):
        for b in range(2):
            blk = 2 * j2 + b

            @pl.when(blk + 1 < nblk)
            def _(b=b, blk=blk):
                fetch(1 - b, blk + 1)

            @pl.when(blk < nblk)
            def _(b=b, blk=blk):
                compute(b, blk)
        return 0

    lax.fori_loop(0, (nblk + 1) // 2, pairstep, 0)
    wait_scatters(0)
    wait_scatters(1)
    plsc.subcore_barrier()

    for t in range(nper // EBA):
        fill_ids(sid * nper + t * EBA)
        pltpu.sync_copy(h_sh.at[id_v], st_va)
        pltpu.sync_copy(
            st_va, hpart_h.at[pl.ds(cid * npad + sid * nper + t * EBA, EBA)])

    def dden(k, _):
        fill_ids(sid * eper + k * EBA)
        pltpu.sync_copy(den_sh.at[id_v], dn_v)
        pltpu.sync_copy(
            dn_v, denp_h.at[pl.ds(cid * npad * NH + sid * eper + k * EBA, EBA)])
        return 0
    lax.fori_loop(0, eper // EBA, dden, 0)


def kernel(h_x, h_e, edge_index, Wq, bq, Wk, bk, Wv, bv, We, be,
           Woh, boh, Woe, boe):
    N, H = h_x.shape
    E = h_e.shape[0]
    assert H == 128 and E % EB == 0

    NP = 10240
    row = edge_index[0].astype(jnp.int32)
    col = edge_index[1].astype(jnp.int32)

    j = jnp.arange(H)
    perm = 8 * (j % DH) + j // DH
    Wq_p = Wq[:, perm] * 0.25
    bq_p = bq[perm] * 0.25
    Wk_p = Wk[:, perm]
    bk_p = bk[perm]
    Wv_p = Wv[:, perm]
    bv_p = bv[perm]
    We_p = We[:, perm]
    be_p = be[perm]
    Woh_p = Woh[perm, :]
    Woe_p = Woe[perm, :]

    Wqkv = jnp.concatenate([Wq_p, Wk_p, Wv_p], axis=1)
    bqkv = jnp.concatenate([bq_p, bk_p, bv_p]).reshape(1, 384)

    NB = 2000
    f32 = jnp.float32

    qp, kp, vp = pl.pallas_call(
        _qkv_body,
        grid=(N // NB,),
        in_specs=[
            pl.BlockSpec((NB, 128), lambda i: (i, 0)),
            pl.BlockSpec((128, 384), lambda i: (0, 0)),
            pl.BlockSpec((1, 384), lambda i: (0, 0)),
        ],
        out_specs=[pl.BlockSpec((NB, 128), lambda i: (i, 0))] * 3,
        out_shape=[jax.ShapeDtypeStruct((N, 128), f32)] * 3,
    )(h_x, Wqkv, bqkv)

    selmat = (jnp.arange(128)[:, None] // DH
              == jnp.arange(NH)[None, :]).astype(f32)
    Wg = We_p @ selmat
    bg = be_p @ selmat
    egate = pl.pallas_call(
        _gate_body,
        grid=(E // NB,),
        in_specs=[
            pl.BlockSpec((NB, 128), lambda i: (i, 0)),
            pl.BlockSpec((128, NH), lambda i: (0, 0)),
            pl.BlockSpec((1, NH), lambda i: (0, 0)),
        ],
        out_specs=pl.BlockSpec((NB, NH), lambda i: (i, 0)),
        out_shape=jax.ShapeDtypeStruct((E, NH), f32),
    )(h_e, Wg, bg.reshape(1, NH))

    nblk_total = E // EB

    scores_f, wmax = pl.kernel(
        functools.partial(_sddmm_body, nblk_total),
        out_type=(jax.ShapeDtypeStruct((E * NH,), f32),
                  jax.ShapeDtypeStruct((NW * L,), f32)),
        mesh=_sc_mesh(),
        scratch_types=(
            [pltpu.VMEM((EB,), jnp.int32),
             pltpu.VMEM((EB,), jnp.int32),
             pltpu.VMEM((EB, 128), f32),
             pltpu.VMEM((EB, 128), f32),
             pltpu.VMEM((EB * NH,), f32),
             pltpu.VMEM((EB * NH,), f32)] * 2
            + [pltpu.VMEM((L,), f32)]
            + [pltpu.SemaphoreType.DMA] * 4
        ),
    )(qp, kp, row, col, egate.reshape(E * NH))

    oe = pl.pallas_call(
        _oe_body,
        grid=(E // NB,),
        in_specs=[
            pl.BlockSpec((NB, 128), lambda i: (i, 0)),
            pl.BlockSpec((NB, NH), lambda i: (i, 0)),
            pl.BlockSpec((128, 128), lambda i: (0, 0)),
            pl.BlockSpec((1, 128), lambda i: (0, 0)),
            pl.BlockSpec((128, 128), lambda i: (0, 0)),
            pl.BlockSpec((1, 128), lambda i: (0, 0)),
        ],
        out_specs=pl.BlockSpec((NB, 128), lambda i: (i, 0)),
        out_shape=jax.ShapeDtypeStruct((E, 128), f32),
    )(h_e, scores_f.reshape(E, NH), We_p, be_p.reshape(1, 128),
      Woe_p, boe.reshape(1, 128))


    denp1, hpart2 = pl.kernel(
        functools.partial(_accum_body, E // EBA),
        out_type=(jax.ShapeDtypeStruct((NC * NP * NH,), f32),
                  jax.ShapeDtypeStruct((NC * NP, 128), f32)),
        mesh=_sc_mesh(),
        scratch_types=(
            [pltpu.VMEM((EBA * NH,), f32),
             pltpu.VMEM((EBA,), jnp.int32),
             pltpu.VMEM((EBA,), jnp.int32),
             pltpu.VMEM((EBA, 128), f32),
             pltpu.VMEM((EBA, 128), f32),
             pltpu.VMEM((NH, EBA), f32),
             pltpu.VMEM((NH, EBA), jnp.int32)] * 2
            + [pltpu.VMEM((EBA,), jnp.int32),
               pltpu.VMEM((EBA,), f32),
               pltpu.VMEM((NW * L,), f32),
               pltpu.VMEM_SHARED((NP * NH,), f32),
               pltpu.VMEM_SHARED((NP, 128), f32)]
            + [pltpu.SemaphoreType.DMA] * 6
        ),
    )(scores_f, wmax, row, col, vp)

    dp0 = denp1[0:NP * NH].reshape(NP, NH)
    dp1 = denp1[NP * NH:].reshape(NP, NH)
    hp0 = hpart2[0:NP]
    hp1 = hpart2[NP:]

    oh = pl.pallas_call(
        _oh_body,
        grid=(N // NB,),
        in_specs=[
            pl.BlockSpec((NB, 128), lambda i: (i, 0)),
            pl.BlockSpec((NB, 128), lambda i: (i, 0)),
            pl.BlockSpec((NB, NH), lambda i: (i, 0)),
            pl.BlockSpec((NB, NH), lambda i: (i, 0)),
            pl.BlockSpec((128, 128), lambda i: (0, 0)),
            pl.BlockSpec((1, 128), lambda i: (0, 0)),
        ],
        out_specs=pl.BlockSpec((NB, 128), lambda i: (i, 0)),
        out_shape=jax.ShapeDtypeStruct((N, 128), f32),
    )(hp0, hp1, dp0, dp1, Woh_p, boh.reshape(1, 128))

    return (oh, oe)


# speedup vs baseline: 1.0531x; 1.0531x over previous
import functools

import jax
import jax.numpy as jnp
from jax import lax
from jax.experimental import pallas as pl
from jax.experimental.pallas import tpu as pltpu
from jax.experimental.pallas import tpu_sc as plsc

NH = 8
DH = 16
L = 16
NC = 2
NS = 16
NW = NC * NS
EB = 128

_GDN = lax.GatherDimensionNumbers(
    offset_dims=(), collapsed_slice_dims=(0,), start_index_map=(0,))


def _vperm(v, idx):
    return lax.gather(v, idx[:, None], _GDN, (1,),
                      mode=lax.GatherScatterMode.PROMISE_IN_BOUNDS)


def _sc_mesh():
    return plsc.VectorSubcoreMesh(core_axis_name="c", subcore_axis_name="s")


def _wid():
    return lax.axis_index("s") * NC + lax.axis_index("c")


def _qkv_body(x_ref, w_ref, b_ref, q_ref, k_ref, v_ref):
    acc = jnp.dot(x_ref[...], w_ref[...], preferred_element_type=jnp.float32)
    acc = acc + b_ref[...]
    q_ref[...] = acc[:, 0:128]
    k_ref[...] = acc[:, 128:256]
    v_ref[...] = acc[:, 256:384]


def _gate_body(x_ref, w_ref, b_ref, g_ref):
    g_ref[...] = jnp.dot(x_ref[...], w_ref[...],
                         preferred_element_type=jnp.float32) + b_ref[...]


def _oe_body(x_ref, s_ref, we_ref, be_ref, w_ref, b_ref, o_ref):
    ep = jnp.dot(x_ref[...], we_ref[...],
                 preferred_element_type=jnp.float32) + be_ref[...]
    rows = lax.broadcasted_iota(jnp.int32, (NH, 128), 0)
    cols = lax.broadcasted_iota(jnp.int32, (NH, 128), 1)
    expand = (cols // DH == rows).astype(jnp.float32)
    sx = jnp.dot(s_ref[...], expand, preferred_element_type=jnp.float32)
    o_ref[...] = jnp.dot(ep * sx, w_ref[...],
                         preferred_element_type=jnp.float32) + b_ref[...]


def _oh_body(a_ref, b2_ref, d0_ref, d1_ref, w_ref, b_ref, o_ref):
    h = a_ref[...] + b2_ref[...]
    den = d0_ref[...] + d1_ref[...]
    rows = lax.broadcasted_iota(jnp.int32, (NH, 128), 0)
    cols = lax.broadcasted_iota(jnp.int32, (NH, 128), 1)
    expand = (cols // DH == rows).astype(jnp.float32)
    dex = jnp.dot(den, expand, preferred_element_type=jnp.float32)
    dex = jnp.where(dex == 0.0, 1.0, dex)
    o_ref[...] = jnp.dot(h / dex, w_ref[...],
                         preferred_element_type=jnp.float32) + b_ref[...]


def _sddmm_body(nblk_total, qp, kp, row_h, col_h, eg_h, sc_h, wmax_h,
                row_v0, col_v0, q_v0, k_v0, eg_v0, sc_v0,
                row_v1, col_v1, q_v1, k_v1, eg_v1, sc_v1,
                mx_v, semg0, semg1, semw0, semw1):
    wid = _wid()
    lane = lax.iota(jnp.int32, L)
    p8 = lane ^ 8
    p4 = lane ^ 4
    p2 = lane ^ 2
    p1 = lane ^ 1
    lo8 = lane < 8
    lane8 = lane & 7

    nblk = nblk_total // NW + jnp.where(wid < nblk_total % NW, 1, 0)
    bufs = ((row_v0, col_v0, q_v0, k_v0, eg_v0, sc_v0, semg0, semw0),
            (row_v1, col_v1, q_v1, k_v1, eg_v1, sc_v1, semg1, semw1))

    mx_v[...] = jnp.full((L,), -3.4e38, jnp.float32)

    def fetch(b, blk):
        row_v, col_v, q_v, k_v, eg_v, sc_v, semg, semw = bufs[b]
        base = (wid + NW * blk) * EB
        pltpu.sync_copy(row_h.at[pl.ds(base, EB)], row_v)
        pltpu.sync_copy(col_h.at[pl.ds(base, EB)], col_v)
        pltpu.async_copy(qp.at[row_v], q_v, semg)
        pltpu.async_copy(kp.at[col_v], k_v, semg)
        pltpu.sync_copy(eg_h.at[pl.ds(base * NH, EB * NH)], eg_v)

    def compute(b, blk):
        row_v, col_v, q_v, k_v, eg_v, sc_v, semg, semw = bufs[b]
        base = (wid + NW * blk) * EB
        pltpu.make_async_copy(qp.at[row_v], q_v, semg).wait()
        pltpu.make_async_copy(kp.at[col_v], k_v, semg).wait()

        def pair(i, mx):
            mf = []
            for h in range(NH):
                p0 = q_v[2 * i, pl.ds(DH * h, L)] * k_v[2 * i, pl.ds(DH * h, L)]
                p1h = (q_v[2 * i + 1, pl.ds(DH * h, L)]
                       * k_v[2 * i + 1, pl.ds(DH * h, L)])
                a0 = p0 + _vperm(p0, p8)
                a1 = p1h + _vperm(p1h, p8)
                m = jnp.where(lo8, a0, a1)
                m = m + _vperm(m, p4)
                m = m + _vperm(m, p2)
                m = m + _vperm(m, p1)
                mf.append(m)
            s = mf[0]
            for h in range(1, NH):
                s = jnp.where(lane8 == h, mf[h], s)
            s = s * eg_v[pl.ds(L * i, L)]
            sc_v[pl.ds(L * i, L)] = s
            return jnp.maximum(mx, s)

        mx = lax.fori_loop(0, EB // 2, pair, mx_v[...])
        mx_v[...] = mx
        pltpu.async_copy(sc_v, sc_h.at[pl.ds(base * NH, EB * NH)], semw)

    def waitw(b):
        row_v, col_v, q_v, k_v, eg_v, sc_v, semg, semw = bufs[b]
        pltpu.make_async_copy(sc_v, sc_h.at[pl.ds(0, EB * NH)], semw).wait()

    fetch(0, 0)

    def pairstep(j2, _):
        for b in range(2):
            blk = 2 * j2 + b

            @pl.when(blk + 1 < nblk)
            def _(b=b, blk=blk):
                fetch(1 - b, blk + 1)

            @pl.when((blk >= 2) & (blk < nblk))
            def _(b=b):
                waitw(b)

            @pl.when(blk < nblk)
            def _(b=b, blk=blk):
                compute(b, blk)
        return 0

    lax.fori_loop(0, (nblk + 1) // 2, pairstep, 0)
    waitw(0)
    waitw(1)
    pltpu.sync_copy(mx_v, wmax_h.at[pl.ds(wid * L, L)])


def _accum_body(nblk_total, sc_h, wmax_h, row_h, col_h, vp, denp_h, hpart_h,
                sc_v, row_v, col_v, v_v, st_v, exs_v, eidx_v, id_v, dn_v,
                wm_v, den_sh, h_sh, sem0, semh, semd):
    cid = lax.axis_index("c")
    sid = lax.axis_index("s")
    wid = sid * NC + cid
    npad = h_sh.shape[0]
    nper = npad // NS
    eper = nper * NH
    lane = lax.iota(jnp.int32, L)
    p8 = lane ^ 8
    lo8 = lane < 8
    lane8 = lane & 7

    nblk = nblk_total // NW + jnp.where(wid < nblk_total % NW, 1, 0)

    def fill_ids(base):
        for j in range(8):
            id_v[pl.ds(16 * j, L)] = base + 16 * j + lane

    def zbody(i, _):
        for h in range(NH):
            st_v[i, pl.ds(DH * h, L)] = jnp.zeros((L,), jnp.float32)
        return 0
    lax.fori_loop(0, EB, zbody, 0)

    def zd(i, _):
        dn_v[pl.ds(L * i, L)] = jnp.zeros((L,), jnp.float32)
        return 0
    lax.fori_loop(0, EB // L, zd, 0)

    for t in range(nper // EB):
        fill_ids(sid * nper + t * EB)
        pltpu.sync_copy(st_v, h_sh.at[id_v])

    def zden(k, _):
        fill_ids(sid * eper + k * EB)
        pltpu.sync_copy(dn_v, den_sh.at[id_v])
        return 0
    lax.fori_loop(0, eper // EB, zden, 0)
    plsc.subcore_barrier()

    pltpu.sync_copy(wmax_h, wm_v)
    m = wm_v[pl.ds(0, L)]
    for j in range(1, NW):
        m = jnp.maximum(m, wm_v[pl.ds(j * L, L)])
    m = jnp.maximum(m, _vperm(m, p8))

    def block(j, _):
        bi = wid + NW * j
        base = bi * EB

        @pl.when(j > 0)
        def _():
            pltpu.make_async_copy(st_v, h_sh.at[row_v], semh).wait()
            for t in range(NH):
                pltpu.make_async_copy(exs_v.at[t],
                                      den_sh.at[eidx_v.at[t]], semd).wait()

        pltpu.sync_copy(col_h.at[pl.ds(base, EB)], col_v)
        pltpu.sync_copy(row_h.at[pl.ds(base, EB)], row_v)
        cv = pltpu.async_copy(vp.at[col_v], v_v, sem0)
        pltpu.sync_copy(sc_h.at[pl.ds(base * NH, EB * NH)], sc_v)
        cv.wait()

        def grp(g, _):
            rv16 = row_v[pl.ds(L * g, L)]
            for j2 in range(8):
                i = 8 * g + j2
                s = sc_v[pl.ds(L * i, L)]
                attn = jnp.exp(s - m)
                exs_v[g, pl.ds(L * j2, L)] = attn
                r0s = _vperm(rv16, jnp.full((L,), 2 * j2, jnp.int32))
                r1s = _vperm(rv16, jnp.full((L,), 2 * j2 + 1, jnp.int32))
                eidx_v[g, pl.ds(L * j2, L)] = (
                    jnp.where(lo8, r0s, r1s) * NH + lane8)
                for e01 in range(2):
                    for h in range(NH):
                        splat = _vperm(attn,
                                       jnp.full((L,), 8 * e01 + h, jnp.int32))
                        st_v[2 * i + e01, pl.ds(DH * h, L)] = (
                            v_v[2 * i + e01, pl.ds(DH * h, L)] * splat)
            return 0

        lax.fori_loop(0, EB // L, grp, 0)
        pltpu.async_copy(st_v, h_sh.at[row_v], semh, add=True)
        for t in range(NH):
            pltpu.async_copy(exs_v.at[t], den_sh.at[eidx_v.at[t]],
                             semd, add=True)
        return 0

    lax.fori_loop(0, nblk, block, 0)

    @pl.when(nblk > 0)
    def _():
        pltpu.make_async_copy(st_v, h_sh.at[row_v], semh).wait()
        for t in range(NH):
            pltpu.make_async_copy(exs_v.at[t],
                                  den_sh.at[eidx_v.at[t]], semd).wait()
    plsc.subcore_barrier()

    for t in range(nper // EB):
        fill_ids(sid * nper + t * EB)
        pltpu.sync_copy(h_sh.at[id_v], st_v)
        pltpu.sync_copy(
            st_v, hpart_h.at[pl.ds(cid * npad + sid * nper + t * EB, EB)])

    def dden(k, _):
        fill_ids(sid * eper + k * EB)
        pltpu.sync_copy(den_sh.at[id_v], dn_v)
        pltpu.sync_copy(
            dn_v, denp_h.at[pl.ds(cid * npad * NH + sid * eper + k * EB, EB)])
        return 0
    lax.fori_loop(0, eper // EB, dden, 0)


def kernel(h_x, h_e, edge_index, Wq, bq, Wk, bk, Wv, bv, We, be,
           Woh, boh, Woe, boe):
    N, H = h_x.shape
    E = h_e.shape[0]
    assert H == 128 and E % EB == 0

    NP = 10240
    row = edge_index[0].astype(jnp.int32)
    col = edge_index[1].astype(jnp.int32)

    j = jnp.arange(H)
    perm = 8 * (j % DH) + j // DH
    Wq_p = Wq[:, perm] * 0.25
    bq_p = bq[perm] * 0.25
    Wk_p = Wk[:, perm]
    bk_p = bk[perm]
    Wv_p = Wv[:, perm]
    bv_p = bv[perm]
    We_p = We[:, perm]
    be_p = be[perm]
    Woh_p = Woh[perm, :]
    Woe_p = Woe[perm, :]

    Wqkv = jnp.concatenate([Wq_p, Wk_p, Wv_p], axis=1)
    bqkv = jnp.concatenate([bq_p, bk_p, bv_p]).reshape(1, 384)

    NB = 2000
    f32 = jnp.float32

    qp, kp, vp = pl.pallas_call(
        _qkv_body,
        grid=(N // NB,),
        in_specs=[
            pl.BlockSpec((NB, 128), lambda i: (i, 0)),
            pl.BlockSpec((128, 384), lambda i: (0, 0)),
            pl.BlockSpec((1, 384), lambda i: (0, 0)),
        ],
        out_specs=[pl.BlockSpec((NB, 128), lambda i: (i, 0))] * 3,
        out_shape=[jax.ShapeDtypeStruct((N, 128), f32)] * 3,
    )(h_x, Wqkv, bqkv)

    selmat = (jnp.arange(128)[:, None] // DH
              == jnp.arange(NH)[None, :]).astype(f32)
    Wg = We_p @ selmat
    bg = be_p @ selmat
    egate = pl.pallas_call(
        _gate_body,
        grid=(E // NB,),
        in_specs=[
            pl.BlockSpec((NB, 128), lambda i: (i, 0)),
            pl.BlockSpec((128, NH), lambda i: (0, 0)),
            pl.BlockSpec((1, NH), lambda i: (0, 0)),
        ],
        out_specs=pl.BlockSpec((NB, NH), lambda i: (i, 0)),
        out_shape=jax.ShapeDtypeStruct((E, NH), f32),
    )(h_e, Wg, bg.reshape(1, NH))

    nblk_total = E // EB

    scores_f, wmax = pl.kernel(
        functools.partial(_sddmm_body, nblk_total),
        out_type=(jax.ShapeDtypeStruct((E * NH,), f32),
                  jax.ShapeDtypeStruct((NW * L,), f32)),
        mesh=_sc_mesh(),
        scratch_types=(
            [pltpu.VMEM((EB,), jnp.int32),
             pltpu.VMEM((EB,), jnp.int32),
             pltpu.VMEM((EB, 128), f32),
             pltpu.VMEM((EB, 128), f32),
             pltpu.VMEM((EB * NH,), f32),
             pltpu.VMEM((EB * NH,), f32)] * 2
            + [pltpu.VMEM((L,), f32)]
            + [pltpu.SemaphoreType.DMA] * 4
        ),
    )(qp, kp, row, col, egate.reshape(E * NH))

    oe = pl.pallas_call(
        _oe_body,
        grid=(E // NB,),
        in_specs=[
            pl.BlockSpec((NB, 128), lambda i: (i, 0)),
            pl.BlockSpec((NB, NH), lambda i: (i, 0)),
            pl.BlockSpec((128, 128), lambda i: (0, 0)),
            pl.BlockSpec((1, 128), lambda i: (0, 0)),
            pl.BlockSpec((128, 128), lambda i: (0, 0)),
            pl.BlockSpec((1, 128), lambda i: (0, 0)),
        ],
        out_specs=pl.BlockSpec((NB, 128), lambda i: (i, 0)),
        out_shape=jax.ShapeDtypeStruct((E, 128), f32),
    )(h_e, scores_f.reshape(E, NH), We_p, be_p.reshape(1, 128),
      Woe_p, boe.reshape(1, 128))


    denp1, hpart2 = pl.kernel(
        functools.partial(_accum_body, nblk_total),
        out_type=(jax.ShapeDtypeStruct((NC * NP * NH,), f32),
                  jax.ShapeDtypeStruct((NC * NP, 128), f32)),
        mesh=_sc_mesh(),
        scratch_types=[
            pltpu.VMEM((EB * NH,), f32),
            pltpu.VMEM((EB,), jnp.int32),
            pltpu.VMEM((EB,), jnp.int32),
            pltpu.VMEM((EB, 128), f32),
            pltpu.VMEM((EB, 128), f32),
            pltpu.VMEM((NH, EB), f32),
            pltpu.VMEM((NH, EB), jnp.int32),
            pltpu.VMEM((EB,), jnp.int32),
            pltpu.VMEM((EB,), f32),
            pltpu.VMEM((NW * L,), f32),
            pltpu.VMEM_SHARED((NP * NH,), f32),
            pltpu.VMEM_SHARED((NP, 128), f32),
            pltpu.SemaphoreType.DMA,
            pltpu.SemaphoreType.DMA,
            pltpu.SemaphoreType.DMA,
        ],
    )(scores_f, wmax, row, col, vp)

    dp0 = denp1[0:NP * NH].reshape(NP, NH)
    dp1 = denp1[NP * NH:].reshape(NP, NH)
    hp0 = hpart2[0:NP]
    hp1 = hpart2[NP:]

    oh = pl.pallas_call(
        _oh_body,
        grid=(N // NB,),
        in_specs=[
            pl.BlockSpec((NB, 128), lambda i: (i, 0)),
            pl.BlockSpec((NB, 128), lambda i: (i, 0)),
            pl.BlockSpec((NB, NH), lambda i: (i, 0)),
            pl.BlockSpec((NB, NH), lambda i: (i, 0)),
            pl.BlockSpec((128, 128), lambda i: (0, 0)),
            pl.BlockSpec((1, 128), lambda i: (0, 0)),
        ],
        out_specs=pl.BlockSpec((NB, 128), lambda i: (i, 0)),
        out_shape=jax.ShapeDtypeStruct((N, 128), f32),
    )(hp0, hp1, dp0, dp1, Woh_p, boh.reshape(1, 128))

    return (oh, oe)
